# Initial kernel scaffold; baseline (speedup 1.0000x reference)
import jax
import jax.numpy as jnp
from jax.experimental import pallas as pl


def kernel(time_encoding, labels, W1, b1, W2, b2, emb):
    raise NotImplementedError("write your pallas kernel here")


# speedup vs baseline: 1.0583x; 1.0583x over previous
import functools

import jax
import jax.numpy as jnp
from jax import lax
from jax.experimental import pallas as pl
from jax.experimental.pallas import tpu as pltpu
from jax.experimental.pallas import tpu_sc as plsc

_B = 16384
_D_TIME = 512
_D_EMB = 1024


_NW = 32
_CHUNK = 64
_ROWS_PER_W = _B // _NW
_CHUNKS_PER_W = _ROWS_PER_W // _CHUNK


def _sc_gather(labels2d, emb):
    mesh = plsc.VectorSubcoreMesh(core_axis_name="c", subcore_axis_name="s")

    @functools.partial(
        pl.kernel,
        mesh=mesh,
        out_type=jax.ShapeDtypeStruct((_B, _D_EMB), jnp.float32),
        scratch_types=[
            pltpu.VMEM((_CHUNK,), jnp.int32),
            pltpu.VMEM((_CHUNK, _D_EMB), jnp.float32),
            pltpu.SemaphoreType.DMA,
        ],
    )
    def gather_k(idx_hbm, table_hbm, out_hbm, idx_v, rows_v, sem):
        wid = lax.axis_index("s") * 2 + lax.axis_index("c")
        for j in range(_CHUNKS_PER_W):
            chunk_id = wid * _CHUNKS_PER_W + j
            base = wid * _ROWS_PER_W + j * _CHUNK
            pltpu.sync_copy(idx_hbm.at[chunk_id], idx_v)
            pltpu.async_copy(table_hbm.at[idx_v], rows_v, sem).wait()
            pltpu.sync_copy(rows_v, out_hbm.at[pl.ds(base, _CHUNK)])

    return gather_k(labels2d, emb)


_BM = 512


def _mlp_body(x_ref, w1_ref, b1_ref, w2_ref, b2_ref, lab_ref, o_ref):
    h = jnp.dot(x_ref[...], w1_ref[...], preferred_element_type=jnp.float32)
    h = h + b1_ref[...]
    h = h * jax.nn.sigmoid(h)
    y = jnp.dot(h, w2_ref[...], preferred_element_type=jnp.float32)
    o_ref[...] = y + b2_ref[...] + lab_ref[...]


def _tc_mlp(x, W1, b1, W2, b2, lab):
    grid = (_B // _BM,)
    return pl.pallas_call(
        _mlp_body,
        grid=grid,
        in_specs=[
            pl.BlockSpec((_BM, _D_TIME), lambda i: (i, 0)),
            pl.BlockSpec((_D_TIME, _D_EMB), lambda i: (0, 0)),
            pl.BlockSpec((1, _D_EMB), lambda i: (0, 0)),
            pl.BlockSpec((_D_EMB, _D_EMB), lambda i: (0, 0)),
            pl.BlockSpec((1, _D_EMB), lambda i: (0, 0)),
            pl.BlockSpec((_BM, _D_EMB), lambda i: (i, 0)),
        ],
        out_specs=pl.BlockSpec((_BM, _D_EMB), lambda i: (i, 0)),
        out_shape=jax.ShapeDtypeStruct((_B, _D_EMB), jnp.float32),
    )(x, W1, b1, W2, b2, lab)


def kernel(time_encoding, labels, W1, b1, W2, b2, emb):
    labels2d = labels.reshape(_B // _CHUNK, _CHUNK)
    lab = _sc_gather(labels2d, emb)
    return _tc_mlp(
        time_encoding,
        W1,
        b1.reshape(1, _D_EMB),
        W2,
        b2.reshape(1, _D_EMB),
        lab,
    )
